# baseline (speedup 1.0000x reference)
import functools

import jax
import jax.numpy as jnp
from jax import lax
from jax.experimental import pallas as pl
from jax.experimental.pallas import tpu as pltpu
from jax.experimental.pallas import tpu_sc as plsc

B, N, C, K = 4, 2048, 128, 16
BN = B * N
BNK = BN * K
EPS = 1e-5

NC, NS = 2, 16
NW = NC * NS

_INTERPRET = False

TKN = 1024
TB = 8192
TS = TB // K
CHUNK = 256


def _knn_body(pr_ref, pt_ref, j_ref):
    bb = pl.program_id(0)
    pr = pr_ref[...]
    pt = pt_ref[0]
    sqr = jnp.sum(pr * pr, axis=1, keepdims=True)
    sqc = jnp.sum(pt * pt, axis=0, keepdims=True)
    cross = lax.dot_general(pr, pt, (((1,), (0,)), ((), ())),
                            preferred_element_type=jnp.float32)
    d0 = (sqr + sqc) - 2.0 * cross
    H = N // 2
    kidx = lax.broadcasted_iota(jnp.int32, (TKN, K), 1)
    pairidx = lax.broadcasted_iota(jnp.int32, (TKN, H), 1)
    INF = jnp.float32(3.0e38)

    a = d0[:, :H]
    b2 = d0[:, H:]
    e = jnp.minimum(a, b2)
    emax = jnp.maximum(a, b2)
    ecol = jnp.where(a <= b2, pairidx, pairidx + H)

    def step(j, carry):
        e, emax, ecol, jacc = carry
        m = jnp.min(e, axis=1, keepdims=True)
        cand = jnp.where(e == m, ecol, N)
        sel = jnp.min(cand, axis=1, keepdims=True)
        jacc = jnp.where(kidx == j, sel, jacc)
        psel = jnp.bitwise_and(sel, H - 1)
        other = jnp.where(sel < H, sel + H, sel - H)
        cond = pairidx == psel
        e = jnp.where(cond, emax, e)
        emax = jnp.where(cond, INF, emax)
        ecol = jnp.where(cond, other, ecol)
        return e, emax, ecol, jacc

    _, _, _, jacc = lax.fori_loop(
        0, K, step, (e, emax, ecol, jnp.zeros((TKN, K), jnp.int32)))
    j_ref[...] = bb * N + jacc


def _knn(p16, pt):
    return pl.pallas_call(
        _knn_body,
        grid=(B, N // TKN),
        in_specs=[
            pl.BlockSpec((TKN, 16), lambda bb, i: (bb * (N // TKN) + i, 0)),
            pl.BlockSpec((1, 16, N), lambda bb, i: (bb, 0, 0)),
        ],
        out_specs=pl.BlockSpec((TKN, K), lambda bb, i: (bb * (N // TKN) + i, 0)),
        out_shape=jax.ShapeDtypeStruct((BN, K), jnp.int32),
        interpret=_INTERPRET,
    )(p16, pt)


def _acc_stats(i, y, sm_ref, sq_ref):
    rows = y.shape[0]
    ps = jnp.sum(y.reshape(rows // 8, 8, C), axis=0)
    pq = jnp.sum((y * y).reshape(rows // 8, 8, C), axis=0)

    @pl.when(i == 0)
    def _():
        sm_ref[...] = ps
        sq_ref[...] = pq

    @pl.when(i > 0)
    def _():
        sm_ref[...] += ps
        sq_ref[...] += pq


def _stage_body_plain(x_ref, w_ref, b_ref, o_ref, sm_ref, sq_ref, *, prec=None):
    i = pl.program_id(0)
    y = jnp.dot(x_ref[...], w_ref[...], precision=prec,
                preferred_element_type=jnp.float32) + b_ref[...]
    o_ref[...] = y
    _acc_stats(i, y, sm_ref, sq_ref)


def _stage_body_affine(x_ref, w_ref, b_ref, s_ref, t_ref, o_ref, sm_ref, sq_ref):
    i = pl.program_id(0)
    x = x_ref[...] * s_ref[...] + t_ref[...]
    y = jnp.dot(x, w_ref[...],
                preferred_element_type=jnp.float32) + b_ref[...]
    o_ref[...] = y
    _acc_stats(i, y, sm_ref, sq_ref)


def _conv_stage(x, wt, bias, s=None, t=None, tile=1024, prec=None):
    rows, in_w = x.shape
    grid = (rows // tile,)
    specs = [
        pl.BlockSpec((tile, in_w), lambda i: (i, 0)),
        pl.BlockSpec((in_w, C), lambda i: (0, 0)),
        pl.BlockSpec((1, C), lambda i: (0, 0)),
    ]
    args = [x, wt, bias]
    body = functools.partial(_stage_body_plain, prec=prec)
    if s is not None:
        specs += [pl.BlockSpec((1, C), lambda i: (0, 0)),
                  pl.BlockSpec((1, C), lambda i: (0, 0))]
        args += [s, t]
        body = _stage_body_affine
    return pl.pallas_call(
        body,
        grid=grid,
        in_specs=specs,
        out_specs=[pl.BlockSpec((tile, C), lambda i: (i, 0)),
                   pl.BlockSpec((8, C), lambda i: (0, 0)),
                   pl.BlockSpec((8, C), lambda i: (0, 0))],
        out_shape=[jax.ShapeDtypeStruct((rows, C), jnp.float32),
                   jax.ShapeDtypeStruct((8, C), jnp.float32),
                   jax.ShapeDtypeStruct((8, C), jnp.float32)],
        interpret=_INTERPRET,
    )(*args)


def _s2_body(x_ref, s_ref, t_ref, wphi_ref, wpsi_ref, wal_ref,
             bphi_ref, bpsi_ref, bal_ref,
             ophi_ref, otbl_ref, mp_ref, qp_ref, ms_ref, qs_ref,
             ma_ref, qa_ref):
    i = pl.program_id(0)
    x = x_ref[...] * s_ref[...] + t_ref[...]
    yphi = jnp.dot(x, wphi_ref[...],
                   preferred_element_type=jnp.float32) + bphi_ref[...]
    ypsi = jnp.dot(x, wpsi_ref[...],
                   preferred_element_type=jnp.float32) + bpsi_ref[...]
    yal = jnp.dot(x, wal_ref[...],
                  preferred_element_type=jnp.float32) + bal_ref[...]
    ophi_ref[...] = yphi
    otbl_ref[...] = jnp.concatenate([ypsi, yal], axis=1)
    _acc_stats(i, yphi, mp_ref, qp_ref)
    _acc_stats(i, ypsi, ms_ref, qs_ref)
    _acc_stats(i, yal, ma_ref, qa_ref)


def _s2_call(u_top, s_top, t_top, wphi, wpsi, wal, bphi, bpsi, bal, tile=1024):
    wspec = lambda: pl.BlockSpec((C, C), lambda i: (0, 0))
    vspec = lambda: pl.BlockSpec((1, C), lambda i: (0, 0))
    sspec = lambda: pl.BlockSpec((8, C), lambda i: (0, 0))
    stat = lambda: jax.ShapeDtypeStruct((8, C), jnp.float32)
    return pl.pallas_call(
        _s2_body,
        grid=(BN // tile,),
        in_specs=[pl.BlockSpec((tile, C), lambda i: (i, 0)),
                  vspec(), vspec(), wspec(), wspec(), wspec(),
                  vspec(), vspec(), vspec()],
        out_specs=[pl.BlockSpec((tile, C), lambda i: (i, 0)),
                   pl.BlockSpec((tile, 2 * C), lambda i: (i, 0)),
                   sspec(), sspec(), sspec(), sspec(), sspec(), sspec()],
        out_shape=[jax.ShapeDtypeStruct((BN, C), jnp.float32),
                   jax.ShapeDtypeStruct((BN, 2 * C), jnp.float32),
                   stat(), stat(), stat(), stat(), stat(), stat()],
        interpret=_INTERPRET,
    )(u_top, s_top, t_top, wphi, wpsi, wal, bphi, bpsi, bal)


def _fin(sm, sq, g, be, cnt):
    mean = jnp.sum(sm, axis=0) / cnt
    var = jnp.sum(sq, axis=0) / cnt - mean * mean
    s = g * lax.rsqrt(var + EPS)
    t = be - mean * s
    return s.reshape(1, C).astype(jnp.float32), t.reshape(1, C).astype(jnp.float32)


def _sc_chunk(width):
    return 32768 // width


def _sc_gather_body(jf_hbm, tbl_hbm, out_hbm,
                    idx0, idx1, buf0, buf1, sem0, sem1, *, width):
    wid = lax.axis_index("s") * NC + lax.axis_index("c")
    chunk = _sc_chunk(width)
    nch = BNK // NW // chunk
    base = wid * (BNK // NW)
    idxs = (idx0, idx1)
    bufs = (buf0, buf1)
    sems = (sem0, sem1)

    for k in (0, 1):
        pltpu.sync_copy(jf_hbm.at[pl.ds(base + k * chunk, chunk)], idxs[k])
        pltpu.async_copy(tbl_hbm.at[idxs[k]], bufs[k], sems[k])

    def body(c2, _):
        for k in (0, 1):
            cc = c2 * 2 + k
            off = base + cc * chunk
            pltpu.make_async_copy(tbl_hbm.at[idxs[k]], bufs[k], sems[k]).wait()
            pltpu.sync_copy(bufs[k], out_hbm.at[pl.ds(off, chunk)])
            nxt = cc + 2

            @pl.when(nxt < nch)
            def _():
                pltpu.sync_copy(
                    jf_hbm.at[pl.ds(base + nxt * chunk, chunk)], idxs[k])
                pltpu.async_copy(tbl_hbm.at[idxs[k]], bufs[k], sems[k])
        return 0

    lax.fori_loop(0, nch // 2, body, 0)


@functools.cache
def _make_sc_gather(width):
    chunk = _sc_chunk(width)
    return functools.partial(
        pl.kernel,
        out_type=jax.ShapeDtypeStruct((BNK, width), jnp.float32),
        mesh=plsc.VectorSubcoreMesh(core_axis_name="c", subcore_axis_name="s"),
        scratch_types=[pltpu.VMEM((chunk,), jnp.int32),
                       pltpu.VMEM((chunk,), jnp.int32),
                       pltpu.VMEM((chunk, width), jnp.float32),
                       pltpu.VMEM((chunk, width), jnp.float32),
                       pltpu.SemaphoreType.DMA,
                       pltpu.SemaphoreType.DMA],
    )(functools.partial(_sc_gather_body, width=width))


def _sc_gather(jf, tbl):
    return _make_sc_gather(tbl.shape[1])(jf, tbl)


def _uh_tile(qj_ref, q_ref, b_ref):
    qj = qj_ref[...]
    qr = q_ref[...]
    return (qr[None] - qj + b_ref[...][None]).reshape(TB, C)


def _b1_body(qj_ref, q_ref, b_ref, sm_ref, sq_ref):
    i = pl.program_id(0)
    y = _uh_tile(qj_ref, q_ref, b_ref)
    _acc_stats(i, y, sm_ref, sq_ref)


def _b2_body(qj_ref, q_ref, b1_ref, s1_ref, t1_ref,
             w2_ref, b2_ref, o_ref, sm_ref, sq_ref):
    i = pl.program_id(0)
    uh = _uh_tile(qj_ref, q_ref, b1_ref)
    h = jnp.maximum(uh * s1_ref[...] + t1_ref[...], 0.0)
    y = jnp.dot(h, w2_ref[...], preferred_element_type=jnp.float32) + b2_ref[...]
    o_ref[...] = y.astype(o_ref.dtype).reshape(K, TS, C)
    _acc_stats(i, y, sm_ref, sq_ref)


def _b3_body(upos_ref, psig_ref, uphi_ref,
             sd2_ref, td2_ref, sphi_ref, tphi_ref, spsi_ref, tpsi_ref,
             w_ref, b_ref, o_ref, sm_ref, sq_ref):
    i = pl.program_id(0)
    pos = (upos_ref[...].astype(jnp.float32) * sd2_ref[...][None]
           + td2_ref[...][None])
    phih = uphi_ref[...] * sphi_ref[...] + tphi_ref[...]
    psih = psig_ref[...] * spsi_ref[...][None] + tpsi_ref[...][None]
    w = (phih[None] - psih + pos).reshape(TB, C)
    y = jnp.dot(w, w_ref[...], preferred_element_type=jnp.float32) + b_ref[...]
    o_ref[...] = y.astype(o_ref.dtype).reshape(K, TS, C)
    _acc_stats(i, y, sm_ref, sq_ref)


def _b4_body(ug1_ref, s_ref, t_ref, w_ref, b_ref, o_ref, sm_ref, sq_ref):
    i = pl.program_id(0)
    g = jnp.maximum(ug1_ref[...].astype(jnp.float32).reshape(TB, C)
                    * s_ref[...] + t_ref[...], 0.0)
    y = jnp.dot(g, w_ref[...], preferred_element_type=jnp.float32) + b_ref[...]
    o_ref[...] = y.astype(o_ref.dtype).reshape(K, TS, C)
    _acc_stats(i, y, sm_ref, sq_ref)


def _b5_body(ug2_ref, upos_ref, alg_ref,
             sg2_ref, tg2_ref, sd2_ref, td2_ref, sal_ref, tal_ref,
             w_ref, b_ref, o_ref, sm_ref, sq_ref):
    i = pl.program_id(0)
    gh = (ug2_ref[...].astype(jnp.float32) * sg2_ref[...][None]
          + tg2_ref[...][None])
    m = jnp.max(gh, axis=0, keepdims=True)
    e = jnp.exp(gh - m)
    den = jnp.sum(e, axis=0, keepdims=True)
    sm = e / den
    pos = (upos_ref[...].astype(jnp.float32) * sd2_ref[...][None]
           + td2_ref[...][None])
    alh = alg_ref[...] * sal_ref[...][None] + tal_ref[...][None]
    y = jnp.sum(sm * (alh + pos), axis=0)
    ud = jnp.dot(y, w_ref[...], preferred_element_type=jnp.float32) + b_ref[...]
    o_ref[...] = ud
    _acc_stats(i, ud, sm_ref, sq_ref)


def _b6_body(ud_ref, x0_ref, s_ref, t_ref, o_ref):
    o_ref[...] = ud_ref[...] * s_ref[...] + t_ref[...] + x0_ref[...]


def _big(body, args, specs, out_rows=None, n_stats=2, tile=TB,
         out_dtype=jnp.float32):
    outs = []
    ospecs = []
    if out_rows == BNK:
        outs.append(jax.ShapeDtypeStruct((K, BN, C), out_dtype))
        ospecs.append(pl.BlockSpec((K, TS, C), lambda i: (0, i, 0)))
    elif out_rows is not None:
        ot = out_rows // (BNK // tile)
        outs.append(jax.ShapeDtypeStruct((out_rows, C), out_dtype))
        ospecs.append(pl.BlockSpec((ot, C), lambda i: (i, 0)))
    for _ in range(n_stats):
        outs.append(jax.ShapeDtypeStruct((8, C), jnp.float32))
        ospecs.append(pl.BlockSpec((8, C), lambda i: (0, 0)))
    return pl.pallas_call(
        body,
        grid=(BNK // tile,),
        in_specs=specs,
        out_specs=ospecs,
        out_shape=outs,
        interpret=_INTERPRET,
    )(*args)


_SPEC_BIG = lambda: pl.BlockSpec((K, TS, C), lambda i: (0, i, 0))
_SPEC_BIG16 = lambda: pl.BlockSpec((TB, 16), lambda i: (i, 0))
_SPEC_SMALL = lambda: pl.BlockSpec((TS, C), lambda i: (i, 0))
_SPEC_SMALL16 = lambda: pl.BlockSpec((TS, 16), lambda i: (i, 0))
_SPEC_W16 = lambda: pl.BlockSpec((16, C), lambda i: (0, 0))
_SPEC_W = lambda: pl.BlockSpec((C, C), lambda i: (0, 0))
_SPEC_V = lambda: pl.BlockSpec((1, C), lambda i: (0, 0))


def kernel(input_p, input_x, a, b, params):
    P = params
    x0 = input_x.transpose(0, 2, 1).reshape(BN, C)
    p3 = input_p.transpose(0, 2, 1).reshape(BN, 3)
    p16 = jnp.pad(p3, ((0, 0), (0, 13)))

    wt_top = P['top_W'].T
    wt_phi = P['phi_W'].T
    wt_psi = P['psi_W'].T
    wt_al = P['al_W'].T
    wt_g1 = P['g1_W'].T
    wt_g2 = P['g2_W'].T
    wt_down = P['down_W'].T
    wt_d1 = jnp.pad(P['d1_W'], ((0, 0), (0, 13))).T
    wt_d2 = P['d2_W'].T
    bias = {k: P[k + '_b'].reshape(1, C) for k in
            ('top', 'phi', 'psi', 'al', 'g1', 'g2', 'down', 'd1', 'd2')}

    pt = jnp.pad(input_p, ((0, 0), (0, 13), (0, 0)))
    jidx = _knn(p16, pt)
    jf = jidx.T.reshape(BNK)

    qpos, _, _ = _conv_stage(p16, wt_d1, jnp.zeros((1, C), jnp.float32),
                             prec=lax.Precision.HIGHEST)

    qj = _sc_gather(jf, qpos).reshape(K, BN, C)

    u_top, m1, q1 = _conv_stage(x0, wt_top, bias['top'])
    s_top, t_top = _fin(m1, q1, P['top_g'], P['top_be'], BN)
    u_phi, tbl, mp, qp, ms, qs, ma, qa = _s2_call(
        u_top, s_top, t_top, wt_phi, wt_psi, wt_al,
        bias['phi'], bias['psi'], bias['al'])
    s_phi, t_phi = _fin(mp, qp, P['phi_g'], P['phi_be'], BN)
    s_psi, t_psi = _fin(ms, qs, P['psi_g'], P['psi_be'], BN)
    s_al, t_al = _fin(ma, qa, P['al_g'], P['al_be'], BN)

    gath = _sc_gather(jf, tbl).reshape(K, BN, 2 * C)
    _QJ = lambda: pl.BlockSpec((K, TS, C), lambda i: (0, i, 0))
    _PSIG = lambda: pl.BlockSpec((K, TS, C), lambda i: (0, i, 0))
    _ALG = lambda: pl.BlockSpec((K, TS, C), lambda i: (0, i, 1))

    md1, qd1 = _big(_b1_body,
                    [qj, qpos, bias['d1']],
                    [_QJ(), _SPEC_SMALL(), _SPEC_V()],
                    out_rows=None)
    s_d1, t_d1 = _fin(md1, qd1, P['d1_g'], P['d1_be'], BNK)

    upos, md2, qd2 = _big(_b2_body,
                          [qj, qpos, bias['d1'], s_d1, t_d1,
                           wt_d2, bias['d2']],
                          [_QJ(), _SPEC_SMALL(), _SPEC_V(),
                           _SPEC_V(), _SPEC_V(), _SPEC_W(),
                           _SPEC_V()],
                          out_rows=BNK, out_dtype=jnp.bfloat16)
    s_d2, t_d2 = _fin(md2, qd2, P['d2_g'], P['d2_be'], BNK)

    ug1, mg1, qg1 = _big(_b3_body,
                         [upos, gath, u_phi,
                          s_d2, t_d2, s_phi, t_phi, s_psi, t_psi,
                          wt_g1, bias['g1']],
                         [_SPEC_BIG(), _PSIG(), _SPEC_SMALL(),
                          _SPEC_V(), _SPEC_V(), _SPEC_V(), _SPEC_V(),
                          _SPEC_V(), _SPEC_V(), _SPEC_W(), _SPEC_V()],
                         out_rows=BNK, out_dtype=jnp.bfloat16)
    s_g1, t_g1 = _fin(mg1, qg1, P['g1_g'], P['g1_be'], BNK)

    ug2, mg2, qg2 = _big(_b4_body,
                         [ug1, s_g1, t_g1, wt_g2, bias['g2']],
                         [_SPEC_BIG(), _SPEC_V(), _SPEC_V(), _SPEC_W(),
                          _SPEC_V()],
                         out_rows=BNK, out_dtype=jnp.bfloat16)
    s_g2, t_g2 = _fin(mg2, qg2, P['g2_g'], P['g2_be'], BNK)

    udown, mdn, qdn = _big(_b5_body,
                           [ug2, upos, gath,
                            s_g2, t_g2, s_d2, t_d2, s_al, t_al,
                            wt_down, bias['down']],
                           [_SPEC_BIG(), _SPEC_BIG(), _ALG(),
                            _SPEC_V(), _SPEC_V(), _SPEC_V(), _SPEC_V(),
                            _SPEC_V(), _SPEC_V(), _SPEC_W(), _SPEC_V()],
                           out_rows=BN)
    s_dn, t_dn = _fin(mdn, qdn, P['down_g'], P['down_be'], BN)

    out = pl.pallas_call(
        _b6_body,
        grid=(BN // 1024,),
        in_specs=[pl.BlockSpec((1024, C), lambda i: (i, 0)),
                  pl.BlockSpec((1024, C), lambda i: (i, 0)),
                  _SPEC_V(), _SPEC_V()],
        out_specs=pl.BlockSpec((1024, C), lambda i: (i, 0)),
        out_shape=jax.ShapeDtypeStruct((BN, C), jnp.float32),
        interpret=_INTERPRET,
    )(udown, x0, s_dn, t_dn)

    return out.reshape(B, N, C).transpose(0, 2, 1)


# speedup vs baseline: 1.0141x; 1.0141x over previous
import functools

import jax
import jax.numpy as jnp
from jax import lax
from jax.experimental import pallas as pl
from jax.experimental.pallas import tpu as pltpu
from jax.experimental.pallas import tpu_sc as plsc

B, N, C, K = 4, 2048, 128, 16
BN = B * N
BNK = BN * K
EPS = 1e-5

NC, NS = 2, 16
NW = NC * NS

_INTERPRET = False

TKN = 1024
TB = 8192
TS = TB // K
CHUNK = 256


def _knn_body(pr_ref, pt_ref, j_ref):
    bb = pl.program_id(0)
    pr = pr_ref[...]
    pt = pt_ref[0]
    sqr = jnp.sum(pr * pr, axis=1, keepdims=True)
    sqc = jnp.sum(pt * pt, axis=0, keepdims=True)
    cross = lax.dot_general(pr, pt, (((1,), (0,)), ((), ())),
                            preferred_element_type=jnp.float32)
    d0 = (sqr + sqc) - 2.0 * cross
    colidx = lax.broadcasted_iota(jnp.int32, (TKN, N), 1)
    kidx = lax.broadcasted_iota(jnp.int32, (TKN, K), 1)

    def step(j, carry):
        d, jacc = carry
        m = jnp.min(d, axis=1, keepdims=True)
        cand = jnp.where(d == m, colidx, N)
        sel = jnp.min(cand, axis=1, keepdims=True)
        jacc = jnp.where(kidx == j, sel, jacc)
        d = jnp.where(colidx == sel, jnp.float32(3.0e38), d)
        return d, jacc

    _, jacc = lax.fori_loop(0, K, step,
                            (d0, jnp.zeros((TKN, K), jnp.int32)))
    j_ref[...] = bb * N + jacc


def _knn(p16, pt):
    return pl.pallas_call(
        _knn_body,
        grid=(B, N // TKN),
        in_specs=[
            pl.BlockSpec((TKN, 16), lambda bb, i: (bb * (N // TKN) + i, 0)),
            pl.BlockSpec((1, 16, N), lambda bb, i: (bb, 0, 0)),
        ],
        out_specs=pl.BlockSpec((TKN, K), lambda bb, i: (bb * (N // TKN) + i, 0)),
        out_shape=jax.ShapeDtypeStruct((BN, K), jnp.int32),
        interpret=_INTERPRET,
    )(p16, pt)


def _acc_stats(i, y, sm_ref, sq_ref):
    rows = y.shape[0]
    ps = jnp.sum(y.reshape(rows // 8, 8, C), axis=0)
    pq = jnp.sum((y * y).reshape(rows // 8, 8, C), axis=0)

    @pl.when(i == 0)
    def _():
        sm_ref[...] = ps
        sq_ref[...] = pq

    @pl.when(i > 0)
    def _():
        sm_ref[...] += ps
        sq_ref[...] += pq


def _stage_body_plain(x_ref, w_ref, b_ref, o_ref, sm_ref, sq_ref, *, prec=None):
    i = pl.program_id(0)
    y = jnp.dot(x_ref[...], w_ref[...], precision=prec,
                preferred_element_type=jnp.float32) + b_ref[...]
    o_ref[...] = y
    _acc_stats(i, y, sm_ref, sq_ref)


def _stage_body_affine(x_ref, w_ref, b_ref, s_ref, t_ref, o_ref, sm_ref, sq_ref):
    i = pl.program_id(0)
    x = x_ref[...] * s_ref[...] + t_ref[...]
    y = jnp.dot(x, w_ref[...],
                preferred_element_type=jnp.float32) + b_ref[...]
    o_ref[...] = y
    _acc_stats(i, y, sm_ref, sq_ref)


def _conv_stage(x, wt, bias, s=None, t=None, tile=1024, prec=None):
    rows, in_w = x.shape
    grid = (rows // tile,)
    specs = [
        pl.BlockSpec((tile, in_w), lambda i: (i, 0)),
        pl.BlockSpec((in_w, C), lambda i: (0, 0)),
        pl.BlockSpec((1, C), lambda i: (0, 0)),
    ]
    args = [x, wt, bias]
    body = functools.partial(_stage_body_plain, prec=prec)
    if s is not None:
        specs += [pl.BlockSpec((1, C), lambda i: (0, 0)),
                  pl.BlockSpec((1, C), lambda i: (0, 0))]
        args += [s, t]
        body = _stage_body_affine
    return pl.pallas_call(
        body,
        grid=grid,
        in_specs=specs,
        out_specs=[pl.BlockSpec((tile, C), lambda i: (i, 0)),
                   pl.BlockSpec((8, C), lambda i: (0, 0)),
                   pl.BlockSpec((8, C), lambda i: (0, 0))],
        out_shape=[jax.ShapeDtypeStruct((rows, C), jnp.float32),
                   jax.ShapeDtypeStruct((8, C), jnp.float32),
                   jax.ShapeDtypeStruct((8, C), jnp.float32)],
        interpret=_INTERPRET,
    )(*args)


def _s2_body(x_ref, s_ref, t_ref, wphi_ref, wpsi_ref, wal_ref,
             bphi_ref, bpsi_ref, bal_ref,
             ophi_ref, otbl_ref, mp_ref, qp_ref, ms_ref, qs_ref,
             ma_ref, qa_ref):
    i = pl.program_id(0)
    x = x_ref[...] * s_ref[...] + t_ref[...]
    yphi = jnp.dot(x, wphi_ref[...],
                   preferred_element_type=jnp.float32) + bphi_ref[...]
    ypsi = jnp.dot(x, wpsi_ref[...],
                   preferred_element_type=jnp.float32) + bpsi_ref[...]
    yal = jnp.dot(x, wal_ref[...],
                  preferred_element_type=jnp.float32) + bal_ref[...]
    ophi_ref[...] = yphi
    otbl_ref[...] = jnp.concatenate([ypsi, yal], axis=1)
    _acc_stats(i, yphi, mp_ref, qp_ref)
    _acc_stats(i, ypsi, ms_ref, qs_ref)
    _acc_stats(i, yal, ma_ref, qa_ref)


def _s2_call(u_top, s_top, t_top, wphi, wpsi, wal, bphi, bpsi, bal, tile=1024):
    wspec = lambda: pl.BlockSpec((C, C), lambda i: (0, 0))
    vspec = lambda: pl.BlockSpec((1, C), lambda i: (0, 0))
    sspec = lambda: pl.BlockSpec((8, C), lambda i: (0, 0))
    stat = lambda: jax.ShapeDtypeStruct((8, C), jnp.float32)
    return pl.pallas_call(
        _s2_body,
        grid=(BN // tile,),
        in_specs=[pl.BlockSpec((tile, C), lambda i: (i, 0)),
                  vspec(), vspec(), wspec(), wspec(), wspec(),
                  vspec(), vspec(), vspec()],
        out_specs=[pl.BlockSpec((tile, C), lambda i: (i, 0)),
                   pl.BlockSpec((tile, 2 * C), lambda i: (i, 0)),
                   sspec(), sspec(), sspec(), sspec(), sspec(), sspec()],
        out_shape=[jax.ShapeDtypeStruct((BN, C), jnp.float32),
                   jax.ShapeDtypeStruct((BN, 2 * C), jnp.float32),
                   stat(), stat(), stat(), stat(), stat(), stat()],
        interpret=_INTERPRET,
    )(u_top, s_top, t_top, wphi, wpsi, wal, bphi, bpsi, bal)


def _fin(sm, sq, g, be, cnt):
    mean = jnp.sum(sm, axis=0) / cnt
    var = jnp.sum(sq, axis=0) / cnt - mean * mean
    s = g * lax.rsqrt(var + EPS)
    t = be - mean * s
    return s.reshape(1, C).astype(jnp.float32), t.reshape(1, C).astype(jnp.float32)


def _sc_chunk(width):
    return 32768 // width


def _sc_gather_body(jf_hbm, tbl_hbm, out_hbm,
                    idx0, idx1, buf0, buf1, sem0, sem1, *, width):
    wid = lax.axis_index("s") * NC + lax.axis_index("c")
    chunk = _sc_chunk(width)
    nch = BNK // NW // chunk
    base = wid * (BNK // NW)
    idxs = (idx0, idx1)
    bufs = (buf0, buf1)
    sems = (sem0, sem1)

    for k in (0, 1):
        pltpu.sync_copy(jf_hbm.at[pl.ds(base + k * chunk, chunk)], idxs[k])
        pltpu.async_copy(tbl_hbm.at[idxs[k]], bufs[k], sems[k])

    def body(c2, _):
        for k in (0, 1):
            cc = c2 * 2 + k
            off = base + cc * chunk
            pltpu.make_async_copy(tbl_hbm.at[idxs[k]], bufs[k], sems[k]).wait()
            pltpu.sync_copy(bufs[k], out_hbm.at[pl.ds(off, chunk)])
            nxt = cc + 2

            @pl.when(nxt < nch)
            def _():
                pltpu.sync_copy(
                    jf_hbm.at[pl.ds(base + nxt * chunk, chunk)], idxs[k])
                pltpu.async_copy(tbl_hbm.at[idxs[k]], bufs[k], sems[k])
        return 0

    lax.fori_loop(0, nch // 2, body, 0)


@functools.cache
def _make_sc_gather(width):
    chunk = _sc_chunk(width)
    return functools.partial(
        pl.kernel,
        out_type=jax.ShapeDtypeStruct((BNK, width), jnp.float32),
        mesh=plsc.VectorSubcoreMesh(core_axis_name="c", subcore_axis_name="s"),
        scratch_types=[pltpu.VMEM((chunk,), jnp.int32),
                       pltpu.VMEM((chunk,), jnp.int32),
                       pltpu.VMEM((chunk, width), jnp.float32),
                       pltpu.VMEM((chunk, width), jnp.float32),
                       pltpu.SemaphoreType.DMA,
                       pltpu.SemaphoreType.DMA],
    )(functools.partial(_sc_gather_body, width=width))


def _sc_gather(jf, tbl):
    return _make_sc_gather(tbl.shape[1])(jf, tbl)


def _uh_tile(qj_ref, q_ref, b_ref):
    qj = qj_ref[...]
    qr = q_ref[...]
    return (qr[None] - qj + b_ref[...][None]).reshape(TB, C)


def _b1_body(qj_ref, q_ref, b_ref, sm_ref, sq_ref):
    i = pl.program_id(0)
    y = _uh_tile(qj_ref, q_ref, b_ref)
    _acc_stats(i, y, sm_ref, sq_ref)


def _b2_body(qj_ref, q_ref, b1_ref, s1_ref, t1_ref,
             w2_ref, b2_ref, o_ref, sm_ref, sq_ref):
    i = pl.program_id(0)
    uh = _uh_tile(qj_ref, q_ref, b1_ref)
    h = jnp.maximum(uh * s1_ref[...] + t1_ref[...], 0.0)
    y = jnp.dot(h, w2_ref[...], preferred_element_type=jnp.float32) + b2_ref[...]
    o_ref[...] = y.astype(o_ref.dtype).reshape(K, TS, C)
    _acc_stats(i, y, sm_ref, sq_ref)


def _b3_body(upos_ref, psig_ref, uphi_ref,
             sd2_ref, td2_ref, sphi_ref, tphi_ref, spsi_ref, tpsi_ref,
             w_ref, b_ref, o_ref, sm_ref, sq_ref):
    i = pl.program_id(0)
    pos = (upos_ref[...].astype(jnp.float32) * sd2_ref[...][None]
           + td2_ref[...][None])
    phih = uphi_ref[...] * sphi_ref[...] + tphi_ref[...]
    psih = psig_ref[...] * spsi_ref[...][None] + tpsi_ref[...][None]
    w = (phih[None] - psih + pos).reshape(TB, C)
    y = jnp.dot(w, w_ref[...], preferred_element_type=jnp.float32) + b_ref[...]
    o_ref[...] = y.astype(o_ref.dtype).reshape(K, TS, C)
    _acc_stats(i, y, sm_ref, sq_ref)


def _b4_body(ug1_ref, s_ref, t_ref, w_ref, b_ref, o_ref, sm_ref, sq_ref):
    i = pl.program_id(0)
    g = jnp.maximum(ug1_ref[...].astype(jnp.float32).reshape(TB, C)
                    * s_ref[...] + t_ref[...], 0.0)
    y = jnp.dot(g, w_ref[...], preferred_element_type=jnp.float32) + b_ref[...]
    o_ref[...] = y.astype(o_ref.dtype).reshape(K, TS, C)
    _acc_stats(i, y, sm_ref, sq_ref)


def _b5_body(ug2_ref, upos_ref, alg_ref,
             sg2_ref, tg2_ref, sd2_ref, td2_ref, sal_ref, tal_ref,
             w_ref, b_ref, o_ref, sm_ref, sq_ref):
    i = pl.program_id(0)
    gh = (ug2_ref[...].astype(jnp.float32) * sg2_ref[...][None]
          + tg2_ref[...][None])
    m = jnp.max(gh, axis=0, keepdims=True)
    e = jnp.exp(gh - m)
    den = jnp.sum(e, axis=0, keepdims=True)
    sm = e / den
    pos = (upos_ref[...].astype(jnp.float32) * sd2_ref[...][None]
           + td2_ref[...][None])
    alh = alg_ref[...] * sal_ref[...][None] + tal_ref[...][None]
    y = jnp.sum(sm * (alh + pos), axis=0)
    ud = jnp.dot(y, w_ref[...], preferred_element_type=jnp.float32) + b_ref[...]
    o_ref[...] = ud
    _acc_stats(i, ud, sm_ref, sq_ref)


def _b6_body(ud_ref, x0_ref, s_ref, t_ref, o_ref):
    o_ref[...] = ud_ref[...] * s_ref[...] + t_ref[...] + x0_ref[...]


def _big(body, args, specs, out_rows=None, n_stats=2, tile=TB,
         out_dtype=jnp.float32):
    outs = []
    ospecs = []
    if out_rows == BNK:
        outs.append(jax.ShapeDtypeStruct((K, BN, C), out_dtype))
        ospecs.append(pl.BlockSpec((K, TS, C), lambda i: (0, i, 0)))
    elif out_rows is not None:
        ot = out_rows // (BNK // tile)
        outs.append(jax.ShapeDtypeStruct((out_rows, C), out_dtype))
        ospecs.append(pl.BlockSpec((ot, C), lambda i: (i, 0)))
    for _ in range(n_stats):
        outs.append(jax.ShapeDtypeStruct((8, C), jnp.float32))
        ospecs.append(pl.BlockSpec((8, C), lambda i: (0, 0)))
    return pl.pallas_call(
        body,
        grid=(BNK // tile,),
        in_specs=specs,
        out_specs=ospecs,
        out_shape=outs,
        interpret=_INTERPRET,
    )(*args)


_SPEC_BIG = lambda: pl.BlockSpec((K, TS, C), lambda i: (0, i, 0))
_SPEC_BIG16 = lambda: pl.BlockSpec((TB, 16), lambda i: (i, 0))
_SPEC_SMALL = lambda: pl.BlockSpec((TS, C), lambda i: (i, 0))
_SPEC_SMALL16 = lambda: pl.BlockSpec((TS, 16), lambda i: (i, 0))
_SPEC_W16 = lambda: pl.BlockSpec((16, C), lambda i: (0, 0))
_SPEC_W = lambda: pl.BlockSpec((C, C), lambda i: (0, 0))
_SPEC_V = lambda: pl.BlockSpec((1, C), lambda i: (0, 0))


def kernel(input_p, input_x, a, b, params):
    P = params
    x0 = input_x.transpose(0, 2, 1).reshape(BN, C)
    p3 = input_p.transpose(0, 2, 1).reshape(BN, 3)
    p16 = jnp.pad(p3, ((0, 0), (0, 13)))

    wt_top = P['top_W'].T
    wt_phi = P['phi_W'].T
    wt_psi = P['psi_W'].T
    wt_al = P['al_W'].T
    wt_g1 = P['g1_W'].T
    wt_g2 = P['g2_W'].T
    wt_down = P['down_W'].T
    wt_d1 = jnp.pad(P['d1_W'], ((0, 0), (0, 13))).T
    wt_d2 = P['d2_W'].T
    bias = {k: P[k + '_b'].reshape(1, C) for k in
            ('top', 'phi', 'psi', 'al', 'g1', 'g2', 'down', 'd1', 'd2')}

    pt = jnp.pad(input_p, ((0, 0), (0, 13), (0, 0)))
    jidx = _knn(p16, pt)
    jf = jidx.T.reshape(BNK)

    qpos, _, _ = _conv_stage(p16, wt_d1, jnp.zeros((1, C), jnp.float32),
                             prec=lax.Precision.HIGHEST)

    qj = _sc_gather(jf, qpos).reshape(K, BN, C)

    u_top, m1, q1 = _conv_stage(x0, wt_top, bias['top'])
    s_top, t_top = _fin(m1, q1, P['top_g'], P['top_be'], BN)
    u_phi, tbl, mp, qp, ms, qs, ma, qa = _s2_call(
        u_top, s_top, t_top, wt_phi, wt_psi, wt_al,
        bias['phi'], bias['psi'], bias['al'])
    s_phi, t_phi = _fin(mp, qp, P['phi_g'], P['phi_be'], BN)
    s_psi, t_psi = _fin(ms, qs, P['psi_g'], P['psi_be'], BN)
    s_al, t_al = _fin(ma, qa, P['al_g'], P['al_be'], BN)

    gath = _sc_gather(jf, tbl).reshape(K, BN, 2 * C)
    _QJ = lambda: pl.BlockSpec((K, TS, C), lambda i: (0, i, 0))
    _PSIG = lambda: pl.BlockSpec((K, TS, C), lambda i: (0, i, 0))
    _ALG = lambda: pl.BlockSpec((K, TS, C), lambda i: (0, i, 1))

    md1, qd1 = _big(_b1_body,
                    [qj, qpos, bias['d1']],
                    [_QJ(), _SPEC_SMALL(), _SPEC_V()],
                    out_rows=None)
    s_d1, t_d1 = _fin(md1, qd1, P['d1_g'], P['d1_be'], BNK)

    upos, md2, qd2 = _big(_b2_body,
                          [qj, qpos, bias['d1'], s_d1, t_d1,
                           wt_d2, bias['d2']],
                          [_QJ(), _SPEC_SMALL(), _SPEC_V(),
                           _SPEC_V(), _SPEC_V(), _SPEC_W(),
                           _SPEC_V()],
                          out_rows=BNK, out_dtype=jnp.bfloat16)
    s_d2, t_d2 = _fin(md2, qd2, P['d2_g'], P['d2_be'], BNK)

    ug1, mg1, qg1 = _big(_b3_body,
                         [upos, gath, u_phi,
                          s_d2, t_d2, s_phi, t_phi, s_psi, t_psi,
                          wt_g1, bias['g1']],
                         [_SPEC_BIG(), _PSIG(), _SPEC_SMALL(),
                          _SPEC_V(), _SPEC_V(), _SPEC_V(), _SPEC_V(),
                          _SPEC_V(), _SPEC_V(), _SPEC_W(), _SPEC_V()],
                         out_rows=BNK, out_dtype=jnp.bfloat16)
    s_g1, t_g1 = _fin(mg1, qg1, P['g1_g'], P['g1_be'], BNK)

    ug2, mg2, qg2 = _big(_b4_body,
                         [ug1, s_g1, t_g1, wt_g2, bias['g2']],
                         [_SPEC_BIG(), _SPEC_V(), _SPEC_V(), _SPEC_W(),
                          _SPEC_V()],
                         out_rows=BNK, out_dtype=jnp.bfloat16)
    s_g2, t_g2 = _fin(mg2, qg2, P['g2_g'], P['g2_be'], BNK)

    udown, mdn, qdn = _big(_b5_body,
                           [ug2, upos, gath,
                            s_g2, t_g2, s_d2, t_d2, s_al, t_al,
                            wt_down, bias['down']],
                           [_SPEC_BIG(), _SPEC_BIG(), _ALG(),
                            _SPEC_V(), _SPEC_V(), _SPEC_V(), _SPEC_V(),
                            _SPEC_V(), _SPEC_V(), _SPEC_W(), _SPEC_V()],
                           out_rows=BN)
    s_dn, t_dn = _fin(mdn, qdn, P['down_g'], P['down_be'], BN)

    out = pl.pallas_call(
        _b6_body,
        grid=(BN // 1024,),
        in_specs=[pl.BlockSpec((1024, C), lambda i: (i, 0)),
                  pl.BlockSpec((1024, C), lambda i: (i, 0)),
                  _SPEC_V(), _SPEC_V()],
        out_specs=pl.BlockSpec((1024, C), lambda i: (i, 0)),
        out_shape=jax.ShapeDtypeStruct((BN, C), jnp.float32),
        interpret=_INTERPRET,
    )(udown, x0, s_dn, t_dn)

    return out.reshape(B, N, C).transpose(0, 2, 1)
